# baseline (speedup 1.0000x reference)
import functools

import jax
import jax.numpy as jnp
from jax import lax
from jax.experimental import pallas as pl
from jax.experimental.pallas import tpu as pltpu
from jax.experimental.pallas import tpu_sc as plsc

N = 10000
E = 160000
D = 256
H = 128
NC = 2
NS = 16
EPW = E // NS
NP = 10240
RPW = NP // NS
EPWP = 10240

_mesh = plsc.VectorSubcoreMesh(core_axis_name="c", subcore_axis_name="s")


def _make_sc_agg(K, G, with_counts):
    NCH = EPWP // K
    assert NCH % (2 * G) == 0

    out_type = [
        jax.ShapeDtypeStruct((NP, H), jnp.float32),
        jax.ShapeDtypeStruct((NP, H), jnp.float32),
    ]
    scratch = [
        pltpu.VMEM_SHARED((NP, H), jnp.float32),
        pltpu.VMEM((2, G, K), jnp.int32),
        pltpu.VMEM((2, G, K), jnp.int32),
        pltpu.VMEM((G, K, H), jnp.float32),
        [pltpu.SemaphoreType.DMA] * (2 * G),
        [pltpu.SemaphoreType.DMA] * (2 * G),
        [pltpu.SemaphoreType.DMA] * G,
        [pltpu.SemaphoreType.DMA] * G,
    ]
    if with_counts:
        out_type.append(jax.ShapeDtypeStruct((NS, NP), jnp.float32))
        scratch.append(pltpu.VMEM((NP,), jnp.float32))

    def body(zA, zB, src, dst, zeros, outA, outB, *rest):
        if with_counts:
            (cnt_out, acc, idx_s, idx_d, rows, isems, dsems, gsems, ssems,
             hist) = rest
        else:
            acc, idx_s, idx_d, rows, isems, dsems, gsems, ssems = rest
        c = lax.axis_index("c")
        s = lax.axis_index("s")
        r0 = pl.multiple_of(s * RPW, 8)
        pltpu.sync_copy(zeros.at[pl.ds(r0, RPW)], acc.at[pl.ds(r0, RPW)])
        ones16 = jnp.ones((16,), jnp.float32)
        if with_counts:
            zero16 = jnp.zeros((16,), jnp.float32)

            def zero_step(i, carry):
                hist[pl.ds(i * 16, 16)] = zero16
                return carry

            lax.fori_loop(0, NP // 16, zero_step, 0)
        plsc.subcore_barrier()

        def issue_idx(jo, p, b):
            base = pl.multiple_of(s * EPWP + (jo * G + b) * K, 8)
            pltpu.async_copy(src.at[pl.ds(base, K)], idx_s.at[p, b],
                             isems[p * G + b])
            pltpu.async_copy(dst.at[pl.ds(base, K)], idx_d.at[p, b],
                             dsems[p * G + b])

        def wait_idx_s(p, b):
            pltpu.make_async_copy(src.at[pl.ds(0, K)], idx_s.at[p, b],
                                  isems[p * G + b]).wait()

        def wait_idx_d(p, b):
            pltpu.make_async_copy(dst.at[pl.ds(0, K)], idx_d.at[p, b],
                                  dsems[p * G + b]).wait()

        def wait_scatter(p, b):
            pltpu.make_async_copy(rows.at[b], acc.at[idx_d.at[p, b]],
                                  ssems[b]).wait()

        def make_it(z_ref):
            def it_body(jo, p):
                q = 1 - p
                gds = []
                for b in range(G):
                    @pl.when(jo > 0)
                    def _():
                        wait_scatter(q, b)

                    @pl.when(jo + 1 < NCH // G)
                    def _():
                        issue_idx(jo + 1, q, b)

                    wait_idx_s(p, b)
                    gds.append(pltpu.async_copy(z_ref.at[idx_s.at[p, b]],
                                                rows.at[b], gsems[b]))
                for b in range(G):
                    gds[b].wait()
                    wait_idx_d(p, b)
                    if with_counts:
                        for v in range(K // 16):
                            idx16 = idx_d[p, b, pl.ds(v * 16, 16)]
                            plsc.addupdate_scatter(hist, [idx16], ones16)
                    pltpu.async_copy(rows.at[b], acc.at[idx_d.at[p, b]],
                                     ssems[b], add=True)
            return it_body

        def run(z_ref):
            it_body = make_it(z_ref)
            for b in range(G):
                issue_idx(0, 0, b)

            def outer(jo2, carry):
                it_body(2 * jo2, 0)
                it_body(2 * jo2 + 1, 1)
                return carry

            lax.fori_loop(0, NCH // G // 2, outer, 0)
            for b in range(G):
                wait_scatter(1, b)

        @pl.when(c == 0)
        def _():
            run(zA)

        @pl.when(c == 1)
        def _():
            run(zB)

        plsc.subcore_barrier()

        @pl.when(c == 0)
        def _():
            pltpu.sync_copy(acc.at[pl.ds(r0, RPW)], outA.at[pl.ds(r0, RPW)])
            if with_counts:
                pltpu.sync_copy(hist, cnt_out.at[s])

        @pl.when(c == 1)
        def _():
            pltpu.sync_copy(acc.at[pl.ds(r0, RPW)], outB.at[pl.ds(r0, RPW)])

    params = (pltpu.CompilerParams(needs_layout_passes=False)
              if with_counts else None)
    return pl.kernel(body, out_type=tuple(out_type), mesh=_mesh,
                     scratch_types=scratch, compiler_params=params)


_sc_agg1 = _make_sc_agg(64, 4, True)
_sc_agg = _make_sc_agg(80, 4, False)


R = 1024
GRID = NP // R

_row_spec_h = pl.BlockSpec((R, H), lambda i: (i, 0))
_row_spec_d = pl.BlockSpec((R, D), lambda i: (i, 0))
_cnt_spec = pl.BlockSpec((R, 8), lambda i: (i, 0))
_w_hd = pl.BlockSpec((H, D), lambda i: (0, 0))
_w_dd = pl.BlockSpec((D, D), lambda i: (0, 0))
_b_spec = pl.BlockSpec((1, D), lambda i: (0, 0))
_a_spec = pl.BlockSpec(memory_space=pltpu.SMEM)


def _inv_body(cnt_ref, out_ref):
    t = jnp.transpose(cnt_ref[...])
    cnt = jnp.sum(t, axis=1, keepdims=True)
    inv = 1.0 / jnp.maximum(cnt, 1.0)
    out_ref[...] = jnp.broadcast_to(inv, out_ref.shape)


_tc_inv = pl.pallas_call(
    _inv_body,
    grid=(GRID,),
    in_specs=[pl.BlockSpec((NS, R), lambda i: (0, i))],
    out_specs=pl.BlockSpec((R, 8), lambda i: (i, 0)),
    out_shape=jax.ShapeDtypeStruct((NP, 8), jnp.float32),
)


def _prelu(v, a):
    return jnp.where(v >= 0, v, a * v)


def _mean_term(sA_ref, sB_ref, inv_ref, WlaT_ref, WlbT_ref, bl_ref):
    inv = inv_ref[:, 0:1]
    s = (
        jnp.dot(sA_ref[...], WlaT_ref[...], preferred_element_type=jnp.float32)
        + jnp.dot(sB_ref[...], WlbT_ref[...], preferred_element_type=jnp.float32)
    )
    return s * inv + bl_ref[...]


def _tc1_body(x_ref, sA_ref, sB_ref, inv_ref, WsT_ref, bs_ref,
              WlaT_ref, WlbT_ref, bl_ref, WrT_ref, a_ref, outA_ref, outB_ref):
    a = a_ref[0]
    x = x_ref[...]
    root = jnp.dot(x, WrT_ref[...], preferred_element_type=jnp.float32)
    h1 = _prelu(_mean_term(sA_ref, sB_ref, inv_ref, WlaT_ref,
                           WlbT_ref, bl_ref) + root, a)
    z2 = jnp.dot(x, WsT_ref[...], preferred_element_type=jnp.float32) \
        + bs_ref[...] + h1
    outA_ref[...] = z2[:, :H]
    outB_ref[...] = z2[:, H:]


_tc1 = pl.pallas_call(
    _tc1_body,
    grid=(GRID,),
    in_specs=[_row_spec_d, _row_spec_h, _row_spec_h, _cnt_spec,
              _w_dd, _b_spec, _w_hd, _w_hd, _b_spec, _w_dd, _a_spec],
    out_specs=(_row_spec_h, _row_spec_h),
    out_shape=(
        jax.ShapeDtypeStruct((N, H), jnp.float32),
        jax.ShapeDtypeStruct((N, H), jnp.float32),
    ),
)


def _tc23_body(residual, zA_ref, zB_ref, sA_ref, sB_ref, inv_ref,
               WlaT_ref, WlbT_ref, bl_ref, WraT_ref, WrbT_ref, a_ref,
               *out_refs):
    a = a_ref[0]
    root = (
        jnp.dot(zA_ref[...], WraT_ref[...], preferred_element_type=jnp.float32)
        + jnp.dot(zB_ref[...], WrbT_ref[...], preferred_element_type=jnp.float32)
    )
    h = _prelu(_mean_term(sA_ref, sB_ref, inv_ref, WlaT_ref,
                          WlbT_ref, bl_ref) + root, a)
    if residual:
        out_refs[0][...] = zA_ref[...] + h[:, :H]
        out_refs[1][...] = zB_ref[...] + h[:, H:]
    else:
        out_refs[0][...] = h


def _make_tc23(residual):
    if residual:
        out_specs = (_row_spec_h, _row_spec_h)
        out_shape = (
            jax.ShapeDtypeStruct((N, H), jnp.float32),
            jax.ShapeDtypeStruct((N, H), jnp.float32),
        )
    else:
        out_specs = (_row_spec_d,)
        out_shape = (jax.ShapeDtypeStruct((N, D), jnp.float32),)
    return pl.pallas_call(
        functools.partial(_tc23_body, residual),
        grid=(GRID,),
        in_specs=[_row_spec_h, _row_spec_h, _row_spec_h, _row_spec_h,
                  _cnt_spec, _w_hd, _w_hd, _b_spec, _w_hd,
                  _w_hd, _a_spec],
        out_specs=out_specs,
        out_shape=out_shape,
    )


_tc2 = _make_tc23(True)
_tc3 = _make_tc23(False)


def kernel(x, W_skip, b_skip, Wl1, bl1, Wr1, Wl2, bl2, Wr2, Wl3, bl3, Wr3, a,
           edge_index):
    f32 = jnp.float32
    src = edge_index[0].astype(jnp.int32)
    dst = edge_index[1].astype(jnp.int32)
    pad = ((0, 0), (0, EPWP - EPW))
    srcP = jnp.pad(src.reshape(NS, EPW), pad).reshape(-1)
    dstP = jnp.pad(dst.reshape(NS, EPW), pad,
                   constant_values=NP - 1).reshape(-1)

    xA = x[:, :H]
    xB = x[:, H:]
    zeros = jnp.zeros((NP, H), f32)

    WsT = W_skip.T
    Wr1T = Wr1.T
    bs2 = b_skip.reshape(1, D)
    bl1_2 = bl1.reshape(1, D)
    bl2_2 = bl2.reshape(1, D)
    bl3_2 = bl3.reshape(1, D)
    Wl1aT, Wl1bT = Wl1[:, :H].T, Wl1[:, H:].T
    Wl2aT, Wl2bT = Wl2[:, :H].T, Wl2[:, H:].T
    Wl3aT, Wl3bT = Wl3[:, :H].T, Wl3[:, H:].T
    Wr2aT, Wr2bT = Wr2[:, :H].T, Wr2[:, H:].T
    Wr3aT, Wr3bT = Wr3[:, :H].T, Wr3[:, H:].T
    a1 = a.reshape(1).astype(f32)

    sA, sB, cnt16r = _sc_agg1(xA, xB, srcP, dstP, zeros)
    inv8 = _tc_inv(cnt16r)
    z2A, z2B = _tc1(x, sA, sB, inv8, WsT, bs2, Wl1aT, Wl1bT, bl1_2,
                    Wr1T, a1)

    sA, sB = _sc_agg(z2A, z2B, srcP, dstP, zeros)
    z3A, z3B = _tc2(z2A, z2B, sA, sB, inv8, Wl2aT, Wl2bT, bl2_2, Wr2aT,
                    Wr2bT, a1)

    sA, sB = _sc_agg(z3A, z3B, srcP, dstP, zeros)
    (h3,) = _tc3(z3A, z3B, sA, sB, inv8, Wl3aT, Wl3bT, bl3_2, Wr3aT,
                 Wr3bT, a1)
    return h3
